# baseline (speedup 1.0000x reference)
import functools

import jax
import jax.numpy as jnp
from jax import lax
from jax.experimental import pallas as pl
from jax.experimental.pallas import tpu as pltpu
from jax.experimental.pallas import tpu_sc as plsc

EMBED = 64
NC = 2
NS = 16
NW = NC * NS

CHUNK = 512


@functools.lru_cache(maxsize=None)
def _build(batch, tlen):
    B = batch * tlen
    b_per_w = B // NW
    nchunks = b_per_w // CHUNK
    chunks_per_j = batch // CHUNK
    assert b_per_w % CHUNK == 0 and batch % CHUNK == 0

    mesh = plsc.VectorSubcoreMesh(core_axis_name="c", subcore_axis_name="s")

    @functools.partial(
        pl.kernel,
        mesh=mesh,
        out_type=jax.ShapeDtypeStruct((batch, tlen, 128), jnp.float32),
        compiler_params=pltpu.CompilerParams(use_tc_tiling_on_sc=False),
        scratch_types=[
            pltpu.VMEM((b_per_w,), jnp.int32),
            pltpu.VMEM((CHUNK, EMBED), jnp.float32),
            pltpu.VMEM((CHUNK, EMBED), jnp.float32),
            pltpu.VMEM((CHUNK, EMBED), jnp.float32),
            pltpu.SemaphoreType.DMA,
            pltpu.SemaphoreType.DMA,
            pltpu.SemaphoreType.DMA,
            pltpu.SemaphoreType.DMA,
            pltpu.SemaphoreType.DMA,
            pltpu.SemaphoreType.DMA,
        ],
    )
    def k(table_hbm, idx_hbm, out_hbm, idx_v,
          r0, r1, r2, sg0, sg1, sg2, so0, so1, so2):
        wid = lax.axis_index("s") * NC + lax.axis_index("c")
        base = wid * b_per_w
        pltpu.sync_copy(idx_hbm.at[pl.ds(base, b_per_w)], idx_v)
        rows = (r0, r1, r2)
        gsem = (sg0, sg1, sg2)
        osem = (so0, so1, so2)

        def gather(g, buf):
            off = pl.multiple_of(g * CHUNK, 8)
            return pltpu.async_copy(
                table_hbm.at[idx_v.at[pl.ds(off, CHUNK)]], rows[buf],
                gsem[buf],
            )

        def wait_gather(g, buf):
            off = pl.multiple_of(g * CHUNK, 8)
            pltpu.make_async_copy(
                table_hbm.at[idx_v.at[pl.ds(off, CHUNK)]], rows[buf],
                gsem[buf],
            ).wait()

        def _dst(g):
            c = wid * nchunks + g
            j = c // chunks_per_j
            i0 = (c % chunks_per_j) * CHUNK
            return out_hbm.at[pl.ds(i0, CHUNK), j, pl.ds(0, EMBED)]

        def out_copy(g, buf):
            return pltpu.async_copy(rows[buf], _dst(g), osem[buf])

        def drain_out(g, buf):
            pltpu.make_async_copy(rows[buf], _dst(g), osem[buf]).wait()

        gather(0, 0)

        def body(i, carry):
            for kk in (0, 1, 2):
                g = 3 * i + kk
                b = kk
                nb = (kk + 1) % 3

                @pl.when(g >= 2)
                def _():
                    drain_out(g - 2, nb)

                gather(g + 1, nb)
                wait_gather(g, b)
                out_copy(g, b)
            return carry

        lax.fori_loop(0, (nchunks - 2) // 3, body, 0)
        ga = nchunks - 2
        gb = nchunks - 1
        drain_out(gb - 3, gb % 3)
        gather(gb, gb % 3)
        wait_gather(ga, ga % 3)
        out_copy(ga, ga % 3)
        wait_gather(gb, gb % 3)
        out_copy(gb, gb % 3)
        for g in range(nchunks - 3, nchunks):
            drain_out(g, g % 3)

    return k


@jax.jit
def kernel(x, table):
    b, t = x.shape
    flat = x.T.reshape(b * t)
    outp = _build(b, t)(table, flat)
    return outp[:, :, :EMBED]


# speedup vs baseline: 1.0019x; 1.0019x over previous
import functools

import jax
import jax.numpy as jnp
from jax import lax
from jax.experimental import pallas as pl
from jax.experimental.pallas import tpu as pltpu
from jax.experimental.pallas import tpu_sc as plsc

EMBED = 64
NC = 2
NS = 16
NW = NC * NS

CHUNK = 512


@functools.lru_cache(maxsize=None)
def _build(batch, tlen):
    B = batch * tlen
    b_per_w = B // NW
    nchunks = b_per_w // CHUNK
    chunks_per_j = batch // CHUNK
    assert b_per_w % CHUNK == 0 and batch % CHUNK == 0

    mesh = plsc.VectorSubcoreMesh(core_axis_name="c", subcore_axis_name="s")

    @functools.partial(
        pl.kernel,
        mesh=mesh,
        out_type=jax.ShapeDtypeStruct((batch, tlen, 128), jnp.float32),
        compiler_params=pltpu.CompilerParams(use_tc_tiling_on_sc=False),
        scratch_types=[
            pltpu.VMEM((b_per_w,), jnp.int32),
            pltpu.VMEM((CHUNK, EMBED), jnp.float32),
            pltpu.VMEM((CHUNK, EMBED), jnp.float32),
            pltpu.SemaphoreType.DMA,
            pltpu.SemaphoreType.DMA,
            pltpu.SemaphoreType.DMA,
            pltpu.SemaphoreType.DMA,
        ],
    )
    def k(table_hbm, idx_hbm, out_hbm, idx_v, r0, r1, sg0, sg1, so0, so1):
        wid = lax.axis_index("s") * NC + lax.axis_index("c")
        base = wid * b_per_w
        pltpu.sync_copy(idx_hbm.at[pl.ds(base, b_per_w)], idx_v)
        rows = (r0, r1)
        gsem = (sg0, sg1)
        osem = (so0, so1)

        def gather(g, buf):
            off = pl.multiple_of(g * CHUNK, 8)
            return pltpu.async_copy(
                table_hbm.at[idx_v.at[pl.ds(off, CHUNK)]], rows[buf],
                gsem[buf],
            )

        def _dst(g):
            c = wid * nchunks + g
            j = c // chunks_per_j
            i0 = (c % chunks_per_j) * CHUNK
            return out_hbm.at[pl.ds(i0, CHUNK), j, pl.ds(0, EMBED)]

        def out_copy(g, buf):
            return pltpu.async_copy(rows[buf], _dst(g), osem[buf])

        def drain_out(g, buf):
            pltpu.make_async_copy(rows[buf], _dst(g), osem[buf]).wait()

        def body(i, carry):
            for b in (0, 1):
                g = 2 * i + b

                @pl.when(i > 0)
                def _():
                    drain_out(g, b)

                gather(g, b).wait()
                out_copy(g, b)
            return carry

        lax.fori_loop(0, nchunks // 2, body, 0)
        for b in (0, 1):
            drain_out(nchunks - 2 + b, b)

    return k


@jax.jit
def kernel(x, table):
    b, t = x.shape
    flat = x.T.reshape(b * t)
    outp = _build(b, t)(table, flat)
    return outp[:, :, :EMBED]
